# baseline (speedup 1.0000x reference)
import functools

import jax
import jax.numpy as jnp
from jax.experimental import pallas as pl
from jax.experimental.pallas import tpu as pltpu

_POS_INF_KEY = 0x7F800000
_N_NARROW = 16
_N_CHUNKS = 16
_I32_MAX = 0x7FFFFFFF


def _flat_ids(rows, d, c):
    row_ids = jax.lax.broadcasted_iota(jnp.int32, (rows, d), 0) + c * rows
    lane_ids = jax.lax.broadcasted_iota(jnp.int32, (rows, d), 1)
    return row_ids * d + lane_ids


def _select_body(k, x_ref, out_ref, st_ref, cnt_ref):
    p = pl.program_id(0)
    c = pl.program_id(1)
    n_chunks = pl.num_programs(1)
    rows, d = x_ref.shape

    @pl.when(jnp.logical_and(p == 0, c == 0))
    def _init():
        st_ref[0] = jnp.int32(0)
        st_ref[1] = jnp.int32(_POS_INF_KEY)

    xi = jax.lax.bitcast_convert_type(x_ref[...], jnp.int32)

    @pl.when(p < _N_NARROW)
    def _narrow():
        lo = st_ref[0]
        hi = st_ref[1]
        mid2 = lo + (hi - lo) // 2
        mid1 = lo + (mid2 - lo) // 2
        mid3 = mid2 + (hi - mid2) // 2
        c1 = jnp.sum((xi >= mid1).astype(jnp.int32))
        c2 = jnp.sum((xi >= mid2).astype(jnp.int32))
        c3 = jnp.sum((xi >= mid3).astype(jnp.int32))

        @pl.when(c == 0)
        def _reset():
            cnt_ref[0] = c1
            cnt_ref[1] = c2
            cnt_ref[2] = c3

        @pl.when(c != 0)
        def _acc():
            cnt_ref[0] += c1
            cnt_ref[1] += c2
            cnt_ref[2] += c3

        @pl.when(c == n_chunks - 1)
        def _update():
            t1 = cnt_ref[0] < k
            t2 = cnt_ref[1] < k
            t3 = cnt_ref[2] < k
            st_ref[0] = jnp.where(t1, lo, jnp.where(t2, mid1, jnp.where(t3, mid2, mid3)))
            st_ref[1] = jnp.where(t1, mid1, jnp.where(t2, mid2, jnp.where(t3, mid3, hi)))

    @pl.when(p == _N_NARROW)
    def _count_gt():
        t = st_ref[0]
        cg = jnp.sum((xi >= t + 1).astype(jnp.int32))

        @pl.when(c == 0)
        def _reset():
            cnt_ref[0] = cg

        @pl.when(c != 0)
        def _acc():
            cnt_ref[0] += cg

        @pl.when(c == n_chunks - 1)
        def _store():
            st_ref[2] = cnt_ref[0]
            st_ref[3] = jnp.int32(0)

    @pl.when(p == _N_NARROW + 1)
    def _ties():
        t = st_ref[0]
        r = k - st_ref[2]
        s_prev = st_ref[3]
        eq = xi == t
        c_block = jnp.sum(eq.astype(jnp.int32))
        flat = _flat_ids(rows, d, c)

        @pl.when(jnp.logical_and(s_prev < r, r <= s_prev + c_block))
        def _extract():
            need = r - s_prev

            def body(_, last):
                cand = jnp.where(jnp.logical_and(eq, flat > last), flat, _I32_MAX)
                return jnp.min(cand)

            cutoff = jax.lax.fori_loop(0, need, body, jnp.int32(-1))
            out_ref[0] = t
            out_ref[1] = cutoff

        st_ref[3] = s_prev + c_block


def _mask_body(x_ref, tc_ref, o_ref):
    t = tc_ref[0]
    cut = tc_ref[1]
    rows, d = x_ref.shape
    xs = x_ref[...]
    xi = jax.lax.bitcast_convert_type(xs, jnp.int32)
    flat = _flat_ids(rows, d, pl.program_id(0))
    keep = jnp.logical_or(xi > t, jnp.logical_and(xi == t, flat <= cut))
    o_ref[...] = jnp.where(keep, xs, 0.0)


def _build_calls(b, d, interpret=False):
    k = min(32 * b, b * d)
    n_chunks = min(_N_CHUNKS, b)
    rows = b // n_chunks
    select = pl.pallas_call(
        functools.partial(_select_body, k),
        grid=(_N_NARROW + 2, n_chunks),
        in_specs=[pl.BlockSpec((rows, d), lambda p, c: (c, 0))],
        out_specs=pl.BlockSpec(memory_space=pltpu.SMEM),
        out_shape=jax.ShapeDtypeStruct((2,), jnp.int32),
        scratch_shapes=[pltpu.SMEM((4,), jnp.int32), pltpu.SMEM((3,), jnp.int32)],
        interpret=interpret,
    )
    mask = pl.pallas_call(
        _mask_body,
        grid=(n_chunks,),
        in_specs=[
            pl.BlockSpec((rows, d), lambda c: (c, 0)),
            pl.BlockSpec(memory_space=pltpu.SMEM),
        ],
        out_specs=pl.BlockSpec((rows, d), lambda c: (c, 0)),
        out_shape=jax.ShapeDtypeStruct((b, d), jnp.float32),
        interpret=interpret,
    )
    return select, mask


def kernel(x):
    b, d = x.shape
    select, mask = _build_calls(b, d)
    tc = select(x)
    return mask(x, tc)
